# baseline (speedup 1.0000x reference)
import functools

import jax
import jax.numpy as jnp
from jax import lax
from jax.experimental import pallas as pl
from jax.experimental.pallas import tpu as pltpu
from jax.experimental.pallas import tpu_sc as plsc

B = 1024
P = 32
DP = 64
DM = 128
S = 4096
V = 2048

BT = 256
NBT = B // BT
ST = 512
NST = S // ST
KT = 1024

_F32 = jnp.float32
_BF16 = jnp.bfloat16

NW = 32
RW = S // NW
CU = 256
BT2 = BT
NB2 = NBT


def _ln(x, g, b, eps=1e-5):
    m = jnp.mean(x, axis=-1, keepdims=True)
    v = jnp.mean((x - m) ** 2, axis=-1, keepdims=True)
    return (x - m) / jnp.sqrt(v + eps) * g + b


def _k1_body(act_ref, ms_ref, kp_ref, km_ref,
             attnp_ref, attnm_ref, idxp_ref, idxm_ref, wp_ref, wm_ref,
             actbf_ref, qbf_ref):
    act = act_ref[...]
    actbf_ref[...] = act.astype(_BF16)
    q = act[:, 0:DP]
    for p in range(1, P):
        q = q + act[:, p * DP:(p + 1) * DP]
    q = q * (1.0 / P)
    qbf_ref[...] = q.astype(_BF16)

    def head(query, keys, scale, attn_ref, idx_ref, w_ref):
        s = lax.dot_general(query.astype(_BF16), keys.astype(_BF16),
                            (((1,), (1,)), ((), ())),
                            preferred_element_type=_F32)
        s = s * scale
        m = jnp.max(s, axis=1, keepdims=True)
        e = jnp.exp(s - m)
        denom = jnp.sum(e, axis=1, keepdims=True)
        attn = e / denom
        attn_ref[...] = attn.astype(_BF16)
        amax = 1.0 / denom
        surprise = 1.0 - amax
        w = 0.1 * jax.nn.sigmoid(surprise)
        w_ref[...] = w.reshape(1, 1, BT)
        ii = lax.broadcasted_iota(jnp.int32, (BT, S), 1)
        sel = jnp.where(s == m, ii, jnp.int32(2**30))
        idx = jnp.min(sel, axis=1)
        idx_ref[...] = idx.reshape(1, 1, BT)

    head(q, kp_ref[...], 1.0 / (DP ** 0.5), attnp_ref, idxp_ref, wp_ref)
    head(ms_ref[...], km_ref[...], 1.0 / (DM ** 0.5),
         attnm_ref, idxm_ref, wm_ref)


def _k1(actual, macro_state, kp, km):
    out_shape = [
        jax.ShapeDtypeStruct((B, S), _BF16),
        jax.ShapeDtypeStruct((B, S), _BF16),
        jax.ShapeDtypeStruct((NBT, 1, BT), jnp.int32),
        jax.ShapeDtypeStruct((NBT, 1, BT), jnp.int32),
        jax.ShapeDtypeStruct((NBT, 1, BT), _F32),
        jax.ShapeDtypeStruct((NBT, 1, BT), _F32),
        jax.ShapeDtypeStruct((B, V), _BF16),
        jax.ShapeDtypeStruct((B, DP), _BF16),
    ]
    return pl.pallas_call(
        _k1_body,
        grid=(NBT,),
        in_specs=[
            pl.BlockSpec((BT, V), lambda i: (i, 0)),
            pl.BlockSpec((BT, DM), lambda i: (i, 0)),
            pl.BlockSpec((S, DP), lambda i: (0, 0)),
            pl.BlockSpec((S, DM), lambda i: (0, 0)),
        ],
        out_specs=[
            pl.BlockSpec((BT, S), lambda i: (i, 0)),
            pl.BlockSpec((BT, S), lambda i: (i, 0)),
            pl.BlockSpec((1, 1, BT), lambda i: (i, 0, 0)),
            pl.BlockSpec((1, 1, BT), lambda i: (i, 0, 0)),
            pl.BlockSpec((1, 1, BT), lambda i: (i, 0, 0)),
            pl.BlockSpec((1, 1, BT), lambda i: (i, 0, 0)),
            pl.BlockSpec((BT, V), lambda i: (i, 0)),
            pl.BlockSpec((BT, DP), lambda i: (i, 0)),
        ],
        out_shape=out_shape,
    )(actual, macro_state, kp, km)


def _k2_body(ap_ref, am_ref, vp_ref, vm_ref, rp_ref, rm_ref, accp, accm):
    k = pl.program_id(1)
    cp = jnp.dot(ap_ref[...].astype(_F32), vp_ref[...],
                 preferred_element_type=_F32)
    cm = jnp.dot(am_ref[...].astype(_F32), vm_ref[...],
                 preferred_element_type=_F32)

    @pl.when(k == 0)
    def _():
        accp[...] = cp
        accm[...] = cm

    @pl.when(k != 0)
    def _():
        accp[...] += cp
        accm[...] += cm

    @pl.when(k == S // KT - 1)
    def _():
        rp_ref[...] = accp[...].astype(_BF16)
        rm_ref[...] = accm[...].astype(_BF16)


def _k2(attn_p, attn_m, vp, vm):
    return pl.pallas_call(
        _k2_body,
        grid=(NBT, S // KT),
        in_specs=[
            pl.BlockSpec((BT, KT), lambda i, k: (i, k)),
            pl.BlockSpec((BT, KT), lambda i, k: (i, k)),
            pl.BlockSpec((KT, V), lambda i, k: (k, 0)),
            pl.BlockSpec((KT, V), lambda i, k: (k, 0)),
        ],
        out_specs=[
            pl.BlockSpec((BT, V), lambda i, k: (i, 0)),
            pl.BlockSpec((BT, V), lambda i, k: (i, 0)),
        ],
        out_shape=[jax.ShapeDtypeStruct((B, V), _BF16),
                   jax.ShapeDtypeStruct((B, V), _BF16)],
        scratch_shapes=[pltpu.VMEM((BT, V), _F32), pltpu.VMEM((BT, V), _F32)],
    )(attn_p, attn_m, vp, vm)


def _k3_body(rp_ref, rm_ref, w1a_ref, w1b_ref, b1_ref, g_ref, be_ref, o_ref,
             acc):
    k = pl.program_id(1)
    c = jnp.dot(rp_ref[...].astype(_F32), w1a_ref[...],
                preferred_element_type=_F32)
    c = c + jnp.dot(rm_ref[...].astype(_F32), w1b_ref[...],
                    preferred_element_type=_F32)

    @pl.when(k == 0)
    def _():
        acc[...] = c

    @pl.when(k != 0)
    def _():
        acc[...] += c

    @pl.when(k == V // KT - 1)
    def _():
        h = acc[...] + b1_ref[...]
        h = _ln(h, g_ref[...], be_ref[...])
        h = 0.5 * h * (1.0 + lax.erf(h * (2.0 ** -0.5)))
        o_ref[...] = h.astype(_BF16)


def _k3(rp, rm, w1a, w1b, b1, g, be):
    return pl.pallas_call(
        _k3_body,
        grid=(NBT, V // KT),
        in_specs=[
            pl.BlockSpec((BT, KT), lambda i, k: (i, k)),
            pl.BlockSpec((BT, KT), lambda i, k: (i, k)),
            pl.BlockSpec((KT, V), lambda i, k: (k, 0)),
            pl.BlockSpec((KT, V), lambda i, k: (k, 0)),
            pl.BlockSpec((1, V), lambda i, k: (0, 0)),
            pl.BlockSpec((1, V), lambda i, k: (0, 0)),
            pl.BlockSpec((1, V), lambda i, k: (0, 0)),
        ],
        out_specs=pl.BlockSpec((BT, V), lambda i, k: (i, 0)),
        out_shape=jax.ShapeDtypeStruct((B, V), _BF16),
        scratch_shapes=[pltpu.VMEM((BT, V), _F32)],
    )(rp, rm, w1a, w1b, b1, g, be)


def _k4_body(h_ref, w2_ref, b2_ref, ps_ref, pow_ref, pob_ref, pog_ref,
             pobe_ref, o_ref):
    fused = jnp.dot(h_ref[...].astype(_F32), w2_ref[...],
                    preferred_element_type=_F32)
    fused = fused + b2_ref[...]
    for p in range(P):
        xp = jnp.concatenate(
            [ps_ref[:, p, :], fused[:, p * DP:(p + 1) * DP]], axis=1)
        e = jnp.dot(xp, pow_ref[p], preferred_element_type=_F32)
        e = e + pob_ref[p:p + 1, :]
        e = _ln(e, pog_ref[p:p + 1, :], pobe_ref[p:p + 1, :])
        o_ref[:, p, :] = e


def _k4(h, w2, b2, ps, po_w, po_b, po_g, po_beta):
    return pl.pallas_call(
        _k4_body,
        grid=(NBT,),
        in_specs=[
            pl.BlockSpec((BT, V), lambda i: (i, 0)),
            pl.BlockSpec((V, V), lambda i: (0, 0)),
            pl.BlockSpec((1, V), lambda i: (0, 0)),
            pl.BlockSpec((BT, P, DP), lambda i: (i, 0, 0)),
            pl.BlockSpec((P, 2 * DP, DP), lambda i: (0, 0, 0)),
            pl.BlockSpec((P, DP), lambda i: (0, 0)),
            pl.BlockSpec((P, DP), lambda i: (0, 0)),
            pl.BlockSpec((P, DP), lambda i: (0, 0)),
        ],
        out_specs=pl.BlockSpec((BT, P, DP), lambda i: (i, 0, 0)),
        out_shape=jax.ShapeDtypeStruct((B, P, DP), _F32),
    )(h, w2, b2, ps, po_w, po_b, po_g, po_beta)


def _k5_body(idxp_ref, idxm_ref, wp_ref, wm_ref, act_ref, q_ref, ms_ref,
             npv_ref, nmv_ref, npk_ref, nmk_ref, dp_ref, dm_ref,
             accpv, accmv, accpk, accmk, dp_acc, dm_acc):
    sblk = pl.program_id(0)
    b = pl.program_id(1)

    rows = sblk * ST + lax.broadcasted_iota(jnp.int32, (ST, BT2), 0)

    def onehot_w(idx_ref, w_ref):
        idx = idx_ref[0]
        w = w_ref[0]
        hit = rows == jnp.broadcast_to(idx, (ST, BT2))
        return jnp.where(hit, jnp.broadcast_to(w, (ST, BT2)), 0.0)

    ap = onehot_w(idxp_ref, wp_ref)
    am = onehot_w(idxm_ref, wm_ref)
    act = act_ref[...]
    qb = q_ref[...]
    msb = ms_ref[...]

    cpv = jnp.dot(ap.astype(_BF16), act, preferred_element_type=_F32)
    cmv = jnp.dot(am.astype(_BF16), act, preferred_element_type=_F32)
    cpk = jnp.dot(ap.astype(_BF16), qb, preferred_element_type=_F32)
    cmk = jnp.dot(am.astype(_BF16), msb, preferred_element_type=_F32)
    cdp = jnp.sum(ap, axis=1, keepdims=True)
    cdm = jnp.sum(am, axis=1, keepdims=True)

    @pl.when(b == 0)
    def _():
        accpv[...] = cpv
        accmv[...] = cmv
        accpk[...] = cpk
        accmk[...] = cmk
        dp_acc[...] = cdp
        dm_acc[...] = cdm

    @pl.when(b != 0)
    def _():
        accpv[...] += cpv
        accmv[...] += cmv
        accpk[...] += cpk
        accmk[...] += cmk
        dp_acc[...] += cdp
        dm_acc[...] += cdm

    @pl.when(b == NB2 - 1)
    def _():
        npv_ref[...] = accpv[...]
        nmv_ref[...] = accmv[...]
        npk_ref[...] = accpk[...]
        nmk_ref[...] = accmk[...]
        dp_ref[...] = jnp.broadcast_to(dp_acc[...], (ST, 16))
        dm_ref[...] = jnp.broadcast_to(dm_acc[...], (ST, 16))


def _k5(idxp, idxm, wp, wm, act_bf, q_bf, ms_bf):
    out_shape = [
        jax.ShapeDtypeStruct((S, V), _F32),
        jax.ShapeDtypeStruct((S, V), _F32),
        jax.ShapeDtypeStruct((S, DP), _F32),
        jax.ShapeDtypeStruct((S, DM), _F32),
        jax.ShapeDtypeStruct((S, 16), _F32),
        jax.ShapeDtypeStruct((S, 16), _F32),
    ]
    return pl.pallas_call(
        _k5_body,
        grid=(NST, NB2),
        in_specs=[
            pl.BlockSpec((1, 1, BT2), lambda s_, b_: (b_, 0, 0)),
            pl.BlockSpec((1, 1, BT2), lambda s_, b_: (b_, 0, 0)),
            pl.BlockSpec((1, 1, BT2), lambda s_, b_: (b_, 0, 0)),
            pl.BlockSpec((1, 1, BT2), lambda s_, b_: (b_, 0, 0)),
            pl.BlockSpec((BT2, V), lambda s_, b_: (b_, 0)),
            pl.BlockSpec((BT2, DP), lambda s_, b_: (b_, 0)),
            pl.BlockSpec((BT2, DM), lambda s_, b_: (b_, 0)),
        ],
        out_specs=[
            pl.BlockSpec((ST, V), lambda s_, b_: (s_, 0)),
            pl.BlockSpec((ST, V), lambda s_, b_: (s_, 0)),
            pl.BlockSpec((ST, DP), lambda s_, b_: (s_, 0)),
            pl.BlockSpec((ST, DM), lambda s_, b_: (s_, 0)),
            pl.BlockSpec((ST, 16), lambda s_, b_: (s_, 0)),
            pl.BlockSpec((ST, 16), lambda s_, b_: (s_, 0)),
        ],
        out_shape=out_shape,
        scratch_shapes=[
            pltpu.VMEM((ST, V), _F32),
            pltpu.VMEM((ST, V), _F32),
            pltpu.VMEM((ST, DP), _F32),
            pltpu.VMEM((ST, DM), _F32),
            pltpu.VMEM((ST, 1), _F32),
            pltpu.VMEM((ST, 1), _F32),
        ],
    )(idxp, idxm, wp, wm, act_bf, q_bf, ms_bf)


def _make_sc_update():
    mesh = plsc.VectorSubcoreMesh(core_axis_name="c", subcore_axis_name="s")

    @functools.partial(
        pl.kernel,
        out_type=[jax.ShapeDtypeStruct((S, DP), _F32),
                  jax.ShapeDtypeStruct((S, V), _F32),
                  jax.ShapeDtypeStruct((S, DM), _F32),
                  jax.ShapeDtypeStruct((S, V), _F32)],
        mesh=mesh,
        scratch_types=[
            pltpu.VMEM((RW, 16), _F32),
            pltpu.VMEM((RW, 16), _F32),
            pltpu.VMEM((64, CU), _F32),
            pltpu.VMEM((64, CU), _F32),
            pltpu.VMEM((64, CU), _F32),
            pltpu.VMEM((64, CU), _F32),
            pltpu.VMEM((64, DP), _F32),
            pltpu.VMEM((64, DP), _F32),
            pltpu.VMEM((64, DM), _F32),
            pltpu.VMEM((64, DM), _F32),
            pltpu.SemaphoreType.DMA,
            pltpu.SemaphoreType.DMA,
            pltpu.SemaphoreType.DMA,
            pltpu.SemaphoreType.DMA,
        ],
    )
    def sc_update(denp_hbm, npk_hbm, npv_hbm, denm_hbm, nmk_hbm, nmv_hbm,
                  pk_hbm, pv_hbm, mk_hbm, mv_hbm,
                  newpk_hbm, newpv_hbm, newmk_hbm, newmv_hbm,
                  dtp, dtm, bufV0, bufV1, bufN0, bufN1,
                  bufKP, bufNKP, bufKM, bufNKM,
                  lsem0, lsem1, ssem0, ssem1):
        sid = lax.axis_index("s")
        cid = lax.axis_index("c")
        wid = sid * 2 + cid
        r0 = wid * RW
        pltpu.sync_copy(denp_hbm.at[pl.ds(r0, RW)], dtp)
        pltpu.sync_copy(denm_hbm.at[pl.ds(r0, RW)], dtm)

        for keys_hbm, nk_hbm, newk_hbm, dt, kd, bufK, bufNK in (
                (pk_hbm, npk_hbm, newpk_hbm, dtp, DP, bufKP, bufNKP),
                (mk_hbm, nmk_hbm, newmk_hbm, dtm, DM, bufKM, bufNKM)):
            for sb in range(RW // 64):
                rsb = r0 + sb * 64
                pltpu.sync_copy(keys_hbm.at[pl.ds(rsb, 64)], bufK)
                pltpu.sync_copy(nk_hbm.at[pl.ds(rsb, 64)], bufNK)

                def krow(r, c, _sb=sb, _dt=dt, _kd=kd, _bK=bufK, _bNK=bufNK):
                    sc = 1.0 - _dt[_sb * 64 + r, :]
                    for j in range(_kd // 16):
                        sl = pl.ds(j * 16, 16)
                        _bK[r, sl] = _bK[r, sl] * sc + _bNK[r, sl]
                    return c
                lax.fori_loop(0, 64, krow, 0)
                pltpu.sync_copy(bufK, newk_hbm.at[pl.ds(rsb, 64)])

        blocks = []
        for vals_hbm, nv_hbm, newv_hbm, dt in (
                (pv_hbm, npv_hbm, newpv_hbm, dtp),
                (mv_hbm, nmv_hbm, newmv_hbm, dtm)):
            for sb in range(RW // 64):
                for cb in range(V // CU):
                    blocks.append((vals_hbm, nv_hbm, newv_hbm, dt, sb, cb))

        bufV = (bufV0, bufV1)
        bufN = (bufN0, bufN1)
        lsem = (lsem0, lsem1)
        ssem = (ssem0, ssem1)

        def start_load(i, par):
            vals_hbm, nv_hbm, _, _, sb, cb = blocks[i]
            rsb = r0 + sb * 64
            c0 = cb * CU
            dv = pltpu.async_copy(
                vals_hbm.at[pl.ds(rsb, 64), pl.ds(c0, CU)], bufV[par],
                lsem[par])
            dn = pltpu.async_copy(
                nv_hbm.at[pl.ds(rsb, 64), pl.ds(c0, CU)], bufN[par],
                lsem[par])
            return (dv, dn)

        loads = start_load(0, 0)
        stores = [None, None]
        for i, (vals_hbm, nv_hbm, newv_hbm, dt, sb, cb) in enumerate(blocks):
            par = i % 2
            nxt = par ^ 1
            if i + 1 < len(blocks):
                if stores[nxt] is not None:
                    stores[nxt].wait()
                next_loads = start_load(i + 1, nxt)
            loads[0].wait()
            loads[1].wait()

            def vrow(r, c, _sb=sb, _dt=dt, _bV=bufV[par], _bN=bufN[par]):
                sc = 1.0 - _dt[_sb * 64 + r, :]
                for j in range(CU // 16):
                    sl = pl.ds(j * 16, 16)
                    _bV[r, sl] = _bV[r, sl] * sc + _bN[r, sl]
                return c
            lax.fori_loop(0, 64, vrow, 0)

            rsb = r0 + sb * 64
            c0 = cb * CU
            stores[par] = pltpu.async_copy(
                bufV[par], newv_hbm.at[pl.ds(rsb, 64), pl.ds(c0, CU)],
                ssem[par])
            if i + 1 < len(blocks):
                loads = next_loads
        for st in stores:
            if st is not None:
                st.wait()

    return sc_update


_sc_update_both = _make_sc_update()


def kernel(pair_states, macro_state, W1, b1, ln1_g, ln1_b, W2, b2,
           po_W, po_b, po_g, po_beta,
           pair_mem_keys, pair_mem_vals, macro_mem_keys, macro_mem_vals):
    actual = pair_states.reshape(B, V)

    (attn_p, attn_m, idxp, idxm, wp, wm, act_bf, q_bf) = _k1(
        actual, macro_state, pair_mem_keys, macro_mem_keys)

    npv, nmv, npk, nmk, den_p, den_m = _k5(
        idxp, idxm, wp, wm, act_bf, q_bf, macro_state.astype(_BF16))

    new_pk, new_pv, new_mk, new_mv = _sc_update_both(
        den_p, npk, npv, den_m, nmk, nmv,
        pair_mem_keys, pair_mem_vals, macro_mem_keys, macro_mem_vals)

    rp, rm = _k2(attn_p, attn_m, pair_mem_vals, macro_mem_vals)

    h = _k3(rp, rm, W1[:V], W1[V:],
            b1.reshape(1, V), ln1_g.reshape(1, V), ln1_b.reshape(1, V))

    enriched = _k4(h, W2, b2.reshape(1, V), pair_states,
                   po_W, po_b, po_g, po_beta)

    return (enriched, new_pk, new_pv, new_mk, new_mv)


# speedup vs baseline: 1.1784x; 1.1784x over previous
import functools

import jax
import jax.numpy as jnp
from jax import lax
from jax.experimental import pallas as pl
from jax.experimental.pallas import tpu as pltpu
from jax.experimental.pallas import tpu_sc as plsc

B = 1024
P = 32
DP = 64
DM = 128
S = 4096
V = 2048

BT = 256
NBT = B // BT
ST = 512
NST = S // ST
KT = 1024

_F32 = jnp.float32
_BF16 = jnp.bfloat16

NW = 32
RW = S // NW
CU = 256
BT2 = BT
NB2 = NBT


def _ln(x, g, b, eps=1e-5):
    m = jnp.mean(x, axis=-1, keepdims=True)
    v = jnp.mean((x - m) ** 2, axis=-1, keepdims=True)
    return (x - m) / jnp.sqrt(v + eps) * g + b


def _k1_body(act_ref, ms_ref, kp_ref, km_ref,
             attnp_ref, attnm_ref, idxp_ref, idxm_ref, wp_ref, wm_ref,
             actbf_ref, qbf_ref):
    act = act_ref[...]
    actbf_ref[...] = act.astype(_BF16)
    q = act[:, 0:DP]
    for p in range(1, P):
        q = q + act[:, p * DP:(p + 1) * DP]
    q = q * (1.0 / P)
    qbf_ref[...] = q.astype(_BF16)

    def head(query, keys, scale, attn_ref, idx_ref, w_ref):
        s = lax.dot_general(query.astype(_BF16), keys.astype(_BF16),
                            (((1,), (1,)), ((), ())),
                            preferred_element_type=_F32)
        s = s * scale
        m = jnp.max(s, axis=1, keepdims=True)
        e = jnp.exp(s - m)
        denom = jnp.sum(e, axis=1, keepdims=True)
        attn = e / denom
        attn_ref[...] = attn.astype(_BF16)
        amax = 1.0 / denom
        surprise = 1.0 - amax
        w = 0.1 * jax.nn.sigmoid(surprise)
        w_ref[...] = w.reshape(1, 1, BT)
        ii = lax.broadcasted_iota(jnp.int32, (BT, S), 1)
        sel = jnp.where(s == m, ii, jnp.int32(2**30))
        idx = jnp.min(sel, axis=1)
        idx_ref[...] = idx.reshape(1, 1, BT)

    head(q, kp_ref[...], 1.0 / (DP ** 0.5), attnp_ref, idxp_ref, wp_ref)
    head(ms_ref[...], km_ref[...], 1.0 / (DM ** 0.5),
         attnm_ref, idxm_ref, wm_ref)


def _k1(actual, macro_state, kp, km):
    out_shape = [
        jax.ShapeDtypeStruct((B, S), _BF16),
        jax.ShapeDtypeStruct((B, S), _BF16),
        jax.ShapeDtypeStruct((NBT, 1, BT), jnp.int32),
        jax.ShapeDtypeStruct((NBT, 1, BT), jnp.int32),
        jax.ShapeDtypeStruct((NBT, 1, BT), _F32),
        jax.ShapeDtypeStruct((NBT, 1, BT), _F32),
        jax.ShapeDtypeStruct((B, V), _BF16),
        jax.ShapeDtypeStruct((B, DP), _BF16),
    ]
    return pl.pallas_call(
        _k1_body,
        grid=(NBT,),
        in_specs=[
            pl.BlockSpec((BT, V), lambda i: (i, 0)),
            pl.BlockSpec((BT, DM), lambda i: (i, 0)),
            pl.BlockSpec((S, DP), lambda i: (0, 0)),
            pl.BlockSpec((S, DM), lambda i: (0, 0)),
        ],
        out_specs=[
            pl.BlockSpec((BT, S), lambda i: (i, 0)),
            pl.BlockSpec((BT, S), lambda i: (i, 0)),
            pl.BlockSpec((1, 1, BT), lambda i: (i, 0, 0)),
            pl.BlockSpec((1, 1, BT), lambda i: (i, 0, 0)),
            pl.BlockSpec((1, 1, BT), lambda i: (i, 0, 0)),
            pl.BlockSpec((1, 1, BT), lambda i: (i, 0, 0)),
            pl.BlockSpec((BT, V), lambda i: (i, 0)),
            pl.BlockSpec((BT, DP), lambda i: (i, 0)),
        ],
        out_shape=out_shape,
    )(actual, macro_state, kp, km)


def _mm_body(a_ref, b_ref, o_ref):
    o_ref[...] = jnp.dot(a_ref[...].astype(_F32), b_ref[...],
                         preferred_element_type=_F32).astype(_BF16)


def _k2(attn, vals, n):
    return pl.pallas_call(
        _mm_body,
        grid=(NBT,),
        in_specs=[
            pl.BlockSpec((BT, S), lambda i: (i, 0)),
            pl.BlockSpec((S, n), lambda i: (0, 0)),
        ],
        out_specs=pl.BlockSpec((BT, n), lambda i: (i, 0)),
        out_shape=jax.ShapeDtypeStruct((B, n), _BF16),
    )(attn, vals)


def _k3_body(rp_ref, rm_ref, w1a_ref, w1b_ref, b1_ref, g_ref, be_ref, o_ref):
    h = jnp.dot(rp_ref[...].astype(_F32), w1a_ref[...],
                preferred_element_type=_F32)
    h = h + jnp.dot(rm_ref[...].astype(_F32), w1b_ref[...],
                    preferred_element_type=_F32)
    h = h + b1_ref[...]
    h = _ln(h, g_ref[...], be_ref[...])
    h = 0.5 * h * (1.0 + lax.erf(h * (2.0 ** -0.5)))
    o_ref[...] = h.astype(_BF16)


def _k3(rp, rm, w1a, w1b, b1, g, be):
    return pl.pallas_call(
        _k3_body,
        grid=(NBT,),
        in_specs=[
            pl.BlockSpec((BT, V), lambda i: (i, 0)),
            pl.BlockSpec((BT, V), lambda i: (i, 0)),
            pl.BlockSpec((V, V), lambda i: (0, 0)),
            pl.BlockSpec((V, V), lambda i: (0, 0)),
            pl.BlockSpec((1, V), lambda i: (0, 0)),
            pl.BlockSpec((1, V), lambda i: (0, 0)),
            pl.BlockSpec((1, V), lambda i: (0, 0)),
        ],
        out_specs=pl.BlockSpec((BT, V), lambda i: (i, 0)),
        out_shape=jax.ShapeDtypeStruct((B, V), _BF16),
    )(rp, rm, w1a, w1b, b1, g, be)


def _k4_body(h_ref, w2_ref, b2_ref, ps_ref, pow_ref, pob_ref, pog_ref,
             pobe_ref, o_ref):
    fused = jnp.dot(h_ref[...].astype(_F32), w2_ref[...],
                    preferred_element_type=_F32)
    fused = fused + b2_ref[...]
    for p in range(P):
        xp = jnp.concatenate(
            [ps_ref[:, p, :], fused[:, p * DP:(p + 1) * DP]], axis=1)
        e = jnp.dot(xp, pow_ref[p], preferred_element_type=_F32)
        e = e + pob_ref[p:p + 1, :]
        e = _ln(e, pog_ref[p:p + 1, :], pobe_ref[p:p + 1, :])
        o_ref[:, p, :] = e


def _k4(h, w2, b2, ps, po_w, po_b, po_g, po_beta):
    return pl.pallas_call(
        _k4_body,
        grid=(NBT,),
        in_specs=[
            pl.BlockSpec((BT, V), lambda i: (i, 0)),
            pl.BlockSpec((V, V), lambda i: (0, 0)),
            pl.BlockSpec((1, V), lambda i: (0, 0)),
            pl.BlockSpec((BT, P, DP), lambda i: (i, 0, 0)),
            pl.BlockSpec((P, 2 * DP, DP), lambda i: (0, 0, 0)),
            pl.BlockSpec((P, DP), lambda i: (0, 0)),
            pl.BlockSpec((P, DP), lambda i: (0, 0)),
            pl.BlockSpec((P, DP), lambda i: (0, 0)),
        ],
        out_specs=pl.BlockSpec((BT, P, DP), lambda i: (i, 0, 0)),
        out_shape=jax.ShapeDtypeStruct((B, P, DP), _F32),
    )(h, w2, b2, ps, po_w, po_b, po_g, po_beta)


def _k5_body(idxp_ref, idxm_ref, wp_ref, wm_ref, act_ref, q_ref, ms_ref,
             npv_ref, nmv_ref, npk_ref, nmk_ref, dp_ref, dm_ref,
             accpv, accmv, accpk, accmk, dp_acc, dm_acc):
    sblk = pl.program_id(0)
    b = pl.program_id(1)

    rows = sblk * ST + lax.broadcasted_iota(jnp.int32, (ST, BT2), 0)

    def onehot_w(idx_ref, w_ref):
        idx = idx_ref[0]
        w = w_ref[0]
        hit = rows == jnp.broadcast_to(idx, (ST, BT2))
        return jnp.where(hit, jnp.broadcast_to(w, (ST, BT2)), 0.0)

    ap = onehot_w(idxp_ref, wp_ref)
    am = onehot_w(idxm_ref, wm_ref)
    act = act_ref[...]
    qb = q_ref[...]
    msb = ms_ref[...]

    cpv = jnp.dot(ap.astype(_BF16), act, preferred_element_type=_F32)
    cmv = jnp.dot(am.astype(_BF16), act, preferred_element_type=_F32)
    cpk = jnp.dot(ap.astype(_BF16), qb, preferred_element_type=_F32)
    cmk = jnp.dot(am.astype(_BF16), msb, preferred_element_type=_F32)
    cdp = jnp.sum(ap, axis=1, keepdims=True)
    cdm = jnp.sum(am, axis=1, keepdims=True)

    @pl.when(b == 0)
    def _():
        accpv[...] = cpv
        accmv[...] = cmv
        accpk[...] = cpk
        accmk[...] = cmk
        dp_acc[...] = cdp
        dm_acc[...] = cdm

    @pl.when(b != 0)
    def _():
        accpv[...] += cpv
        accmv[...] += cmv
        accpk[...] += cpk
        accmk[...] += cmk
        dp_acc[...] += cdp
        dm_acc[...] += cdm

    @pl.when(b == NB2 - 1)
    def _():
        npv_ref[...] = accpv[...]
        nmv_ref[...] = accmv[...]
        npk_ref[...] = accpk[...]
        nmk_ref[...] = accmk[...]
        dp_ref[...] = jnp.broadcast_to(dp_acc[...], (ST, 16))
        dm_ref[...] = jnp.broadcast_to(dm_acc[...], (ST, 16))


def _k5(idxp, idxm, wp, wm, act_bf, q_bf, ms_bf):
    out_shape = [
        jax.ShapeDtypeStruct((S, V), _F32),
        jax.ShapeDtypeStruct((S, V), _F32),
        jax.ShapeDtypeStruct((S, DP), _F32),
        jax.ShapeDtypeStruct((S, DM), _F32),
        jax.ShapeDtypeStruct((S, 16), _F32),
        jax.ShapeDtypeStruct((S, 16), _F32),
    ]
    return pl.pallas_call(
        _k5_body,
        grid=(NST, NB2),
        in_specs=[
            pl.BlockSpec((1, 1, BT2), lambda s_, b_: (b_, 0, 0)),
            pl.BlockSpec((1, 1, BT2), lambda s_, b_: (b_, 0, 0)),
            pl.BlockSpec((1, 1, BT2), lambda s_, b_: (b_, 0, 0)),
            pl.BlockSpec((1, 1, BT2), lambda s_, b_: (b_, 0, 0)),
            pl.BlockSpec((BT2, V), lambda s_, b_: (b_, 0)),
            pl.BlockSpec((BT2, DP), lambda s_, b_: (b_, 0)),
            pl.BlockSpec((BT2, DM), lambda s_, b_: (b_, 0)),
        ],
        out_specs=[
            pl.BlockSpec((ST, V), lambda s_, b_: (s_, 0)),
            pl.BlockSpec((ST, V), lambda s_, b_: (s_, 0)),
            pl.BlockSpec((ST, DP), lambda s_, b_: (s_, 0)),
            pl.BlockSpec((ST, DM), lambda s_, b_: (s_, 0)),
            pl.BlockSpec((ST, 16), lambda s_, b_: (s_, 0)),
            pl.BlockSpec((ST, 16), lambda s_, b_: (s_, 0)),
        ],
        out_shape=out_shape,
        scratch_shapes=[
            pltpu.VMEM((ST, V), _F32),
            pltpu.VMEM((ST, V), _F32),
            pltpu.VMEM((ST, DP), _F32),
            pltpu.VMEM((ST, DM), _F32),
            pltpu.VMEM((ST, 1), _F32),
            pltpu.VMEM((ST, 1), _F32),
        ],
    )(idxp, idxm, wp, wm, act_bf, q_bf, ms_bf)


def _make_sc_update():
    mesh = plsc.VectorSubcoreMesh(core_axis_name="c", subcore_axis_name="s")

    @functools.partial(
        pl.kernel,
        out_type=[jax.ShapeDtypeStruct((S, DP), _F32),
                  jax.ShapeDtypeStruct((S, V), _F32),
                  jax.ShapeDtypeStruct((S, DM), _F32),
                  jax.ShapeDtypeStruct((S, V), _F32)],
        mesh=mesh,
        scratch_types=[
            pltpu.VMEM((RW, 16), _F32),
            pltpu.VMEM((RW, 16), _F32),
            pltpu.VMEM((64, CU), _F32),
            pltpu.VMEM((64, CU), _F32),
            pltpu.VMEM((64, CU), _F32),
            pltpu.VMEM((64, CU), _F32),
            pltpu.VMEM((64, DP), _F32),
            pltpu.VMEM((64, DP), _F32),
            pltpu.VMEM((64, DM), _F32),
            pltpu.VMEM((64, DM), _F32),
            pltpu.SemaphoreType.DMA,
            pltpu.SemaphoreType.DMA,
            pltpu.SemaphoreType.DMA,
            pltpu.SemaphoreType.DMA,
        ],
    )
    def sc_update(denp_hbm, npk_hbm, npv_hbm, denm_hbm, nmk_hbm, nmv_hbm,
                  pk_hbm, pv_hbm, mk_hbm, mv_hbm,
                  newpk_hbm, newpv_hbm, newmk_hbm, newmv_hbm,
                  dtp, dtm, bufV0, bufV1, bufN0, bufN1,
                  bufKP, bufNKP, bufKM, bufNKM,
                  lsem0, lsem1, ssem0, ssem1):
        sid = lax.axis_index("s")
        cid = lax.axis_index("c")
        wid = sid * 2 + cid
        r0 = wid * RW
        pltpu.sync_copy(denp_hbm.at[pl.ds(r0, RW)], dtp)
        pltpu.sync_copy(denm_hbm.at[pl.ds(r0, RW)], dtm)

        for keys_hbm, nk_hbm, newk_hbm, dt, kd, bufK, bufNK in (
                (pk_hbm, npk_hbm, newpk_hbm, dtp, DP, bufKP, bufNKP),
                (mk_hbm, nmk_hbm, newmk_hbm, dtm, DM, bufKM, bufNKM)):
            for sb in range(RW // 64):
                rsb = r0 + sb * 64
                pltpu.sync_copy(keys_hbm.at[pl.ds(rsb, 64)], bufK)
                pltpu.sync_copy(nk_hbm.at[pl.ds(rsb, 64)], bufNK)

                def krow(r, c, _sb=sb, _dt=dt, _kd=kd, _bK=bufK, _bNK=bufNK):
                    sc = 1.0 - _dt[_sb * 64 + r, :]
                    for j in range(_kd // 16):
                        sl = pl.ds(j * 16, 16)
                        _bK[r, sl] = _bK[r, sl] * sc + _bNK[r, sl]
                    return c
                lax.fori_loop(0, 64, krow, 0)
                pltpu.sync_copy(bufK, newk_hbm.at[pl.ds(rsb, 64)])

        blocks = []
        for vals_hbm, nv_hbm, newv_hbm, dt in (
                (pv_hbm, npv_hbm, newpv_hbm, dtp),
                (mv_hbm, nmv_hbm, newmv_hbm, dtm)):
            for sb in range(RW // 64):
                for cb in range(V // CU):
                    blocks.append((vals_hbm, nv_hbm, newv_hbm, dt, sb, cb))

        bufV = (bufV0, bufV1)
        bufN = (bufN0, bufN1)
        lsem = (lsem0, lsem1)
        ssem = (ssem0, ssem1)

        def start_load(i, par):
            vals_hbm, nv_hbm, _, _, sb, cb = blocks[i]
            rsb = r0 + sb * 64
            c0 = cb * CU
            dv = pltpu.async_copy(
                vals_hbm.at[pl.ds(rsb, 64), pl.ds(c0, CU)], bufV[par],
                lsem[par])
            dn = pltpu.async_copy(
                nv_hbm.at[pl.ds(rsb, 64), pl.ds(c0, CU)], bufN[par],
                lsem[par])
            return (dv, dn)

        loads = start_load(0, 0)
        stores = [None, None]
        for i, (vals_hbm, nv_hbm, newv_hbm, dt, sb, cb) in enumerate(blocks):
            par = i % 2
            nxt = par ^ 1
            if i + 1 < len(blocks):
                if stores[nxt] is not None:
                    stores[nxt].wait()
                next_loads = start_load(i + 1, nxt)
            loads[0].wait()
            loads[1].wait()

            def vrow(r, c, _sb=sb, _dt=dt, _bV=bufV[par], _bN=bufN[par]):
                sc = 1.0 - _dt[_sb * 64 + r, :]
                for j in range(CU // 16):
                    sl = pl.ds(j * 16, 16)
                    _bV[r, sl] = _bV[r, sl] * sc + _bN[r, sl]
                return c
            lax.fori_loop(0, 64, vrow, 0)

            rsb = r0 + sb * 64
            c0 = cb * CU
            stores[par] = pltpu.async_copy(
                bufV[par], newv_hbm.at[pl.ds(rsb, 64), pl.ds(c0, CU)],
                ssem[par])
            if i + 1 < len(blocks):
                loads = next_loads
        for st in stores:
            if st is not None:
                st.wait()

    return sc_update


_sc_update_both = _make_sc_update()


def kernel(pair_states, macro_state, W1, b1, ln1_g, ln1_b, W2, b2,
           po_W, po_b, po_g, po_beta,
           pair_mem_keys, pair_mem_vals, macro_mem_keys, macro_mem_vals):
    actual = pair_states.reshape(B, V)

    (attn_p, attn_m, idxp, idxm, wp, wm, act_bf, q_bf) = _k1(
        actual, macro_state, pair_mem_keys, macro_mem_keys)

    npv, nmv, npk, nmk, den_p, den_m = _k5(
        idxp, idxm, wp, wm, act_bf, q_bf, macro_state.astype(_BF16))

    new_pk, new_pv, new_mk, new_mv = _sc_update_both(
        den_p, npk, npv, den_m, nmk, nmv,
        pair_mem_keys, pair_mem_vals, macro_mem_keys, macro_mem_vals)

    rp = _k2(attn_p, pair_mem_vals, V)
    rm = _k2(attn_m, macro_mem_vals, V)

    h = _k3(rp, rm, W1[:V], W1[V:],
            b1.reshape(1, V), ln1_g.reshape(1, V), ln1_b.reshape(1, V))

    enriched = _k4(h, W2, b2.reshape(1, V), pair_states,
                   po_W, po_b, po_g, po_beta)

    return (enriched, new_pk, new_pv, new_mk, new_mv)


# speedup vs baseline: 1.2189x; 1.0343x over previous
import functools

import jax
import jax.numpy as jnp
from jax import lax
from jax.experimental import pallas as pl
from jax.experimental.pallas import tpu as pltpu
from jax.experimental.pallas import tpu_sc as plsc

B = 1024
P = 32
DP = 64
DM = 128
S = 4096
V = 2048

BT = 128
NBT = B // BT
ST = 512
NST = S // ST
KT = 1024

_F32 = jnp.float32
_BF16 = jnp.bfloat16

NW = 32
RW = S // NW
CU = 256
BT2 = 256
NB2 = B // BT2


def _ln(x, g, b, eps=1e-5):
    m = jnp.mean(x, axis=-1, keepdims=True)
    v = jnp.mean((x - m) ** 2, axis=-1, keepdims=True)
    return (x - m) / jnp.sqrt(v + eps) * g + b


def _k1_body(act_ref, ms_ref, kp_ref, km_ref,
             attnp_ref, attnm_ref, idxp_ref, idxm_ref, wp_ref, wm_ref,
             actbf_ref, qbf_ref):
    act = act_ref[...]
    actbf_ref[...] = act.astype(_BF16)
    q = act[:, 0:DP]
    for p in range(1, P):
        q = q + act[:, p * DP:(p + 1) * DP]
    q = q * (1.0 / P)
    qbf_ref[...] = q.astype(_BF16)

    def head(query, keys, scale, attn_ref, idx_ref, w_ref):
        s = lax.dot_general(query.astype(_BF16), keys.astype(_BF16),
                            (((1,), (1,)), ((), ())),
                            preferred_element_type=_F32)
        s = s * scale
        m = jnp.max(s, axis=1, keepdims=True)
        e = jnp.exp(s - m)
        denom = jnp.sum(e, axis=1, keepdims=True)
        attn = e / denom
        attn_ref[...] = attn.astype(_BF16)
        amax = 1.0 / denom
        surprise = 1.0 - amax
        w = 0.1 * jax.nn.sigmoid(surprise)
        w_ref[...] = w.reshape(1, 1, BT)
        ii = lax.broadcasted_iota(jnp.int32, (BT, S), 1)
        sel = jnp.where(s == m, ii, jnp.int32(2**30))
        idx = jnp.min(sel, axis=1)
        idx_ref[...] = idx.reshape(1, 1, BT)

    head(q, kp_ref[...], 1.0 / (DP ** 0.5), attnp_ref, idxp_ref, wp_ref)
    head(ms_ref[...], km_ref[...], 1.0 / (DM ** 0.5),
         attnm_ref, idxm_ref, wm_ref)


def _k1(actual, macro_state, kp, km):
    out_shape = [
        jax.ShapeDtypeStruct((B, S), _BF16),
        jax.ShapeDtypeStruct((B, S), _BF16),
        jax.ShapeDtypeStruct((NBT, 1, BT), jnp.int32),
        jax.ShapeDtypeStruct((NBT, 1, BT), jnp.int32),
        jax.ShapeDtypeStruct((NBT, 1, BT), _F32),
        jax.ShapeDtypeStruct((NBT, 1, BT), _F32),
        jax.ShapeDtypeStruct((B, V), _BF16),
        jax.ShapeDtypeStruct((B, DP), _BF16),
    ]
    return pl.pallas_call(
        _k1_body,
        grid=(NBT,),
        in_specs=[
            pl.BlockSpec((BT, V), lambda i: (i, 0)),
            pl.BlockSpec((BT, DM), lambda i: (i, 0)),
            pl.BlockSpec((S, DP), lambda i: (0, 0)),
            pl.BlockSpec((S, DM), lambda i: (0, 0)),
        ],
        out_specs=[
            pl.BlockSpec((BT, S), lambda i: (i, 0)),
            pl.BlockSpec((BT, S), lambda i: (i, 0)),
            pl.BlockSpec((1, 1, BT), lambda i: (i, 0, 0)),
            pl.BlockSpec((1, 1, BT), lambda i: (i, 0, 0)),
            pl.BlockSpec((1, 1, BT), lambda i: (i, 0, 0)),
            pl.BlockSpec((1, 1, BT), lambda i: (i, 0, 0)),
            pl.BlockSpec((BT, V), lambda i: (i, 0)),
            pl.BlockSpec((BT, DP), lambda i: (i, 0)),
        ],
        out_shape=out_shape,
    )(actual, macro_state, kp, km)


def _mm_body(a_ref, b_ref, o_ref):
    o_ref[...] = jnp.dot(a_ref[...].astype(_F32), b_ref[...],
                         preferred_element_type=_F32).astype(_BF16)


def _k2(attn, vals, n):
    return pl.pallas_call(
        _mm_body,
        grid=(NBT,),
        in_specs=[
            pl.BlockSpec((BT, S), lambda i: (i, 0)),
            pl.BlockSpec((S, n), lambda i: (0, 0)),
        ],
        out_specs=pl.BlockSpec((BT, n), lambda i: (i, 0)),
        out_shape=jax.ShapeDtypeStruct((B, n), _BF16),
    )(attn, vals)


def _k3_body(rp_ref, rm_ref, w1a_ref, w1b_ref, b1_ref, g_ref, be_ref, o_ref):
    h = jnp.dot(rp_ref[...].astype(_F32), w1a_ref[...],
                preferred_element_type=_F32)
    h = h + jnp.dot(rm_ref[...].astype(_F32), w1b_ref[...],
                    preferred_element_type=_F32)
    h = h + b1_ref[...]
    h = _ln(h, g_ref[...], be_ref[...])
    h = 0.5 * h * (1.0 + lax.erf(h * (2.0 ** -0.5)))
    o_ref[...] = h.astype(_BF16)


def _k3(rp, rm, w1a, w1b, b1, g, be):
    return pl.pallas_call(
        _k3_body,
        grid=(NBT,),
        in_specs=[
            pl.BlockSpec((BT, V), lambda i: (i, 0)),
            pl.BlockSpec((BT, V), lambda i: (i, 0)),
            pl.BlockSpec((V, V), lambda i: (0, 0)),
            pl.BlockSpec((V, V), lambda i: (0, 0)),
            pl.BlockSpec((1, V), lambda i: (0, 0)),
            pl.BlockSpec((1, V), lambda i: (0, 0)),
            pl.BlockSpec((1, V), lambda i: (0, 0)),
        ],
        out_specs=pl.BlockSpec((BT, V), lambda i: (i, 0)),
        out_shape=jax.ShapeDtypeStruct((B, V), _BF16),
    )(rp, rm, w1a, w1b, b1, g, be)


def _k4_body(h_ref, w2_ref, b2_ref, ps_ref, pow_ref, pob_ref, pog_ref,
             pobe_ref, o_ref):
    fused = jnp.dot(h_ref[...].astype(_F32), w2_ref[...],
                    preferred_element_type=_F32)
    fused = fused + b2_ref[...]
    for p in range(P):
        xp = jnp.concatenate(
            [ps_ref[:, p, :], fused[:, p * DP:(p + 1) * DP]], axis=1)
        e = jnp.dot(xp, pow_ref[p], preferred_element_type=_F32)
        e = e + pob_ref[p:p + 1, :]
        e = _ln(e, pog_ref[p:p + 1, :], pobe_ref[p:p + 1, :])
        o_ref[:, p, :] = e


def _k4(h, w2, b2, ps, po_w, po_b, po_g, po_beta):
    return pl.pallas_call(
        _k4_body,
        grid=(NBT,),
        in_specs=[
            pl.BlockSpec((BT, V), lambda i: (i, 0)),
            pl.BlockSpec((V, V), lambda i: (0, 0)),
            pl.BlockSpec((1, V), lambda i: (0, 0)),
            pl.BlockSpec((BT, P, DP), lambda i: (i, 0, 0)),
            pl.BlockSpec((P, 2 * DP, DP), lambda i: (0, 0, 0)),
            pl.BlockSpec((P, DP), lambda i: (0, 0)),
            pl.BlockSpec((P, DP), lambda i: (0, 0)),
            pl.BlockSpec((P, DP), lambda i: (0, 0)),
        ],
        out_specs=pl.BlockSpec((BT, P, DP), lambda i: (i, 0, 0)),
        out_shape=jax.ShapeDtypeStruct((B, P, DP), _F32),
    )(h, w2, b2, ps, po_w, po_b, po_g, po_beta)


def _k5_body(idxp_ref, idxm_ref, wp_ref, wm_ref, act_ref, q_ref, ms_ref,
             npv_ref, nmv_ref, npk_ref, nmk_ref, dp_ref, dm_ref,
             accpv, accmv, accpk, accmk, dp_acc, dm_acc):
    sblk = pl.program_id(0)
    b = pl.program_id(1)

    rows = sblk * ST + lax.broadcasted_iota(jnp.int32, (ST, BT2), 0)

    def onehot_w(idx_ref, w_ref):
        idx = idx_ref[0]
        w = w_ref[0]
        hit = rows == jnp.broadcast_to(idx, (ST, BT2))
        return jnp.where(hit, jnp.broadcast_to(w, (ST, BT2)), 0.0)

    ap = onehot_w(idxp_ref, wp_ref)
    am = onehot_w(idxm_ref, wm_ref)
    act = act_ref[...]
    qb = q_ref[...]
    msb = ms_ref[...]

    cpv = jnp.dot(ap.astype(_BF16), act, preferred_element_type=_F32)
    cmv = jnp.dot(am.astype(_BF16), act, preferred_element_type=_F32)
    cpk = jnp.dot(ap.astype(_BF16), qb, preferred_element_type=_F32)
    cmk = jnp.dot(am.astype(_BF16), msb, preferred_element_type=_F32)
    cdp = jnp.sum(ap, axis=1, keepdims=True)
    cdm = jnp.sum(am, axis=1, keepdims=True)

    @pl.when(b == 0)
    def _():
        accpv[...] = cpv
        accmv[...] = cmv
        accpk[...] = cpk
        accmk[...] = cmk
        dp_acc[...] = cdp
        dm_acc[...] = cdm

    @pl.when(b != 0)
    def _():
        accpv[...] += cpv
        accmv[...] += cmv
        accpk[...] += cpk
        accmk[...] += cmk
        dp_acc[...] += cdp
        dm_acc[...] += cdm

    @pl.when(b == NB2 - 1)
    def _():
        npv_ref[...] = accpv[...]
        nmv_ref[...] = accmv[...]
        npk_ref[...] = accpk[...]
        nmk_ref[...] = accmk[...]
        dp_ref[...] = jnp.broadcast_to(dp_acc[...], (ST, 16))
        dm_ref[...] = jnp.broadcast_to(dm_acc[...], (ST, 16))


def _k5(idxp, idxm, wp, wm, act_bf, q_bf, ms_bf):
    out_shape = [
        jax.ShapeDtypeStruct((S, V), _F32),
        jax.ShapeDtypeStruct((S, V), _F32),
        jax.ShapeDtypeStruct((S, DP), _F32),
        jax.ShapeDtypeStruct((S, DM), _F32),
        jax.ShapeDtypeStruct((S, 16), _F32),
        jax.ShapeDtypeStruct((S, 16), _F32),
    ]
    return pl.pallas_call(
        _k5_body,
        grid=(NST, NB2),
        in_specs=[
            pl.BlockSpec((1, 1, BT2), lambda s_, b_: (b_, 0, 0)),
            pl.BlockSpec((1, 1, BT2), lambda s_, b_: (b_, 0, 0)),
            pl.BlockSpec((1, 1, BT2), lambda s_, b_: (b_, 0, 0)),
            pl.BlockSpec((1, 1, BT2), lambda s_, b_: (b_, 0, 0)),
            pl.BlockSpec((BT2, V), lambda s_, b_: (b_, 0)),
            pl.BlockSpec((BT2, DP), lambda s_, b_: (b_, 0)),
            pl.BlockSpec((BT2, DM), lambda s_, b_: (b_, 0)),
        ],
        out_specs=[
            pl.BlockSpec((ST, V), lambda s_, b_: (s_, 0)),
            pl.BlockSpec((ST, V), lambda s_, b_: (s_, 0)),
            pl.BlockSpec((ST, DP), lambda s_, b_: (s_, 0)),
            pl.BlockSpec((ST, DM), lambda s_, b_: (s_, 0)),
            pl.BlockSpec((ST, 16), lambda s_, b_: (s_, 0)),
            pl.BlockSpec((ST, 16), lambda s_, b_: (s_, 0)),
        ],
        out_shape=out_shape,
        scratch_shapes=[
            pltpu.VMEM((ST, V), _F32),
            pltpu.VMEM((ST, V), _F32),
            pltpu.VMEM((ST, DP), _F32),
            pltpu.VMEM((ST, DM), _F32),
            pltpu.VMEM((ST, 1), _F32),
            pltpu.VMEM((ST, 1), _F32),
        ],
    )(idxp, idxm, wp, wm, act_bf, q_bf, ms_bf)


def _make_sc_update():
    mesh = plsc.VectorSubcoreMesh(core_axis_name="c", subcore_axis_name="s")

    @functools.partial(
        pl.kernel,
        out_type=[jax.ShapeDtypeStruct((S, DP), _F32),
                  jax.ShapeDtypeStruct((S, V), _F32),
                  jax.ShapeDtypeStruct((S, DM), _F32),
                  jax.ShapeDtypeStruct((S, V), _F32)],
        mesh=mesh,
        scratch_types=[
            pltpu.VMEM((RW, 16), _F32),
            pltpu.VMEM((RW, 16), _F32),
            pltpu.VMEM((64, CU), _F32),
            pltpu.VMEM((64, CU), _F32),
            pltpu.VMEM((64, CU), _F32),
            pltpu.VMEM((64, CU), _F32),
            pltpu.VMEM((64, DP), _F32),
            pltpu.VMEM((64, DP), _F32),
            pltpu.VMEM((64, DM), _F32),
            pltpu.VMEM((64, DM), _F32),
            pltpu.SemaphoreType.DMA,
            pltpu.SemaphoreType.DMA,
            pltpu.SemaphoreType.DMA,
            pltpu.SemaphoreType.DMA,
        ],
    )
    def sc_update(denp_hbm, npk_hbm, npv_hbm, denm_hbm, nmk_hbm, nmv_hbm,
                  pk_hbm, pv_hbm, mk_hbm, mv_hbm,
                  newpk_hbm, newpv_hbm, newmk_hbm, newmv_hbm,
                  dtp, dtm, bufV0, bufV1, bufN0, bufN1,
                  bufKP, bufNKP, bufKM, bufNKM,
                  lsem0, lsem1, ssem0, ssem1):
        sid = lax.axis_index("s")
        cid = lax.axis_index("c")
        wid = sid * 2 + cid
        r0 = wid * RW
        pltpu.sync_copy(denp_hbm.at[pl.ds(r0, RW)], dtp)
        pltpu.sync_copy(denm_hbm.at[pl.ds(r0, RW)], dtm)

        for keys_hbm, nk_hbm, newk_hbm, dt, kd, bufK, bufNK in (
                (pk_hbm, npk_hbm, newpk_hbm, dtp, DP, bufKP, bufNKP),
                (mk_hbm, nmk_hbm, newmk_hbm, dtm, DM, bufKM, bufNKM)):
            for sb in range(RW // 64):
                rsb = r0 + sb * 64
                pltpu.sync_copy(keys_hbm.at[pl.ds(rsb, 64)], bufK)
                pltpu.sync_copy(nk_hbm.at[pl.ds(rsb, 64)], bufNK)

                def krow(r, c, _sb=sb, _dt=dt, _kd=kd, _bK=bufK, _bNK=bufNK):
                    sc = 1.0 - _dt[_sb * 64 + r, :]
                    for j in range(_kd // 16):
                        sl = pl.ds(j * 16, 16)
                        _bK[r, sl] = _bK[r, sl] * sc + _bNK[r, sl]
                    return c
                lax.fori_loop(0, 64, krow, 0)
                pltpu.sync_copy(bufK, newk_hbm.at[pl.ds(rsb, 64)])

        blocks = []
        for vals_hbm, nv_hbm, newv_hbm, dt in (
                (pv_hbm, npv_hbm, newpv_hbm, dtp),
                (mv_hbm, nmv_hbm, newmv_hbm, dtm)):
            for sb in range(RW // 64):
                for cb in range(V // CU):
                    blocks.append((vals_hbm, nv_hbm, newv_hbm, dt, sb, cb))

        bufV = (bufV0, bufV1)
        bufN = (bufN0, bufN1)
        lsem = (lsem0, lsem1)
        ssem = (ssem0, ssem1)

        def start_load(i, par):
            vals_hbm, nv_hbm, _, _, sb, cb = blocks[i]
            rsb = r0 + sb * 64
            c0 = cb * CU
            dv = pltpu.async_copy(
                vals_hbm.at[pl.ds(rsb, 64), pl.ds(c0, CU)], bufV[par],
                lsem[par])
            dn = pltpu.async_copy(
                nv_hbm.at[pl.ds(rsb, 64), pl.ds(c0, CU)], bufN[par],
                lsem[par])
            return (dv, dn)

        loads = start_load(0, 0)
        stores = [None, None]
        for i, (vals_hbm, nv_hbm, newv_hbm, dt, sb, cb) in enumerate(blocks):
            par = i % 2
            nxt = par ^ 1
            if i + 1 < len(blocks):
                if stores[nxt] is not None:
                    stores[nxt].wait()
                next_loads = start_load(i + 1, nxt)
            loads[0].wait()
            loads[1].wait()

            def vrow(r, c, _sb=sb, _dt=dt, _bV=bufV[par], _bN=bufN[par]):
                sc = 1.0 - _dt[_sb * 64 + r, :]
                for j in range(CU // 16):
                    sl = pl.ds(j * 16, 16)
                    _bV[r, sl] = _bV[r, sl] * sc + _bN[r, sl]
                return c
            lax.fori_loop(0, 64, vrow, 0)

            rsb = r0 + sb * 64
            c0 = cb * CU
            stores[par] = pltpu.async_copy(
                bufV[par], newv_hbm.at[pl.ds(rsb, 64), pl.ds(c0, CU)],
                ssem[par])
            if i + 1 < len(blocks):
                loads = next_loads
        for st in stores:
            if st is not None:
                st.wait()

    return sc_update


_sc_update_both = _make_sc_update()


def kernel(pair_states, macro_state, W1, b1, ln1_g, ln1_b, W2, b2,
           po_W, po_b, po_g, po_beta,
           pair_mem_keys, pair_mem_vals, macro_mem_keys, macro_mem_vals):
    actual = pair_states.reshape(B, V)

    (attn_p, attn_m, idxp, idxm, wp, wm, act_bf, q_bf) = _k1(
        actual, macro_state, pair_mem_keys, macro_mem_keys)

    npv, nmv, npk, nmk, den_p, den_m = _k5(
        idxp.reshape(NB2, 1, BT2), idxm.reshape(NB2, 1, BT2),
        wp.reshape(NB2, 1, BT2), wm.reshape(NB2, 1, BT2),
        act_bf, q_bf, macro_state.astype(_BF16))

    new_pk, new_pv, new_mk, new_mv = _sc_update_both(
        den_p, npk, npv, den_m, nmk, nmv,
        pair_mem_keys, pair_mem_vals, macro_mem_keys, macro_mem_vals)

    rp = _k2(attn_p, pair_mem_vals, V)
    rm = _k2(attn_m, macro_mem_vals, V)

    h = _k3(rp, rm, W1[:V], W1[V:],
            b1.reshape(1, V), ln1_g.reshape(1, V), ln1_b.reshape(1, V))

    enriched = _k4(h, W2, b2.reshape(1, V), pair_states,
                   po_W, po_b, po_g, po_beta)

    return (enriched, new_pk, new_pv, new_mk, new_mv)
